# baseline (speedup 1.0000x reference)
import functools

import jax
import jax.numpy as jnp
from jax import lax
from jax.experimental import pallas as pl
from jax.experimental.pallas import tpu as pltpu
from jax.experimental.pallas import tpu_sc as plsc

_N = 2097152
_E = _N * 4
_NW = 32
_RPW = _N // _NW
_EPW = _E // _NW
_CHR = 2048
_CHE = _CHR * 4
_CHUNKS = _RPW // _CHR


def _sc_kernel(o_hbm, t_hbm, l_hbm, tot_hbm, cnt_hbm,
               o_buf, t_buf, l_buf, stage):
    wid = lax.axis_index("s") * 2 + lax.axis_index("c")
    row0 = wid * _RPW
    elem0 = wid * _EPW

    lane = lax.iota(jnp.int32, 16)

    def chunk_body(c, carry):
        acc, cnt = carry
        r0 = row0 + c * _CHR
        pltpu.sync_copy(o_hbm.at[pl.ds(r0, _CHR), :], o_buf)
        pltpu.sync_copy(t_hbm.at[pl.ds(r0, _CHR), :], t_buf)
        pltpu.sync_copy(l_hbm.at[pl.ds(r0, _CHR)], l_buf)

        def group_body(g, carry2):
            acc2, cnt2 = carry2
            gr = g * 16
            l16 = l_buf[pl.ds(gr, 16)]
            msk = jnp.where(l16 == 1, 1.0, 0.0)
            ridx = gr + lane
            hsum = jnp.zeros((16,), jnp.float32)
            for col in range(4):
                cidx = jnp.full((16,), col, jnp.int32)
                o_v = plsc.load_gather(o_buf, [ridx, cidx])
                t_v = plsc.load_gather(t_buf, [ridx, cidx])
                d = o_v - t_v
                a = jnp.abs(d)
                q = jnp.minimum(a, 1.0)
                hsum = hsum + q * (a - 0.5 * q)
            acc2 = acc2 + msk * hsum
            cnt2 = cnt2 + msk
            return acc2, cnt2

        return lax.fori_loop(0, _CHR // 16, group_body, (acc, cnt))

    zeros = jnp.zeros((16,), jnp.float32)
    acc, cnt = lax.fori_loop(0, _CHUNKS, chunk_body, (zeros, zeros))

    stage[...] = acc
    pltpu.sync_copy(stage, tot_hbm.at[wid])
    stage[...] = cnt
    pltpu.sync_copy(stage, cnt_hbm.at[wid])


def _finish_kernel(tot_ref, cnt_ref, out_ref):
    total = jnp.sum(tot_ref[...])
    npos = jnp.sum(cnt_ref[...])
    out_ref[0] = total / jnp.maximum(npos * 4.0, 1.0)


@jax.jit
def kernel(out_bbox, labels, bbox_targets):
    mesh = plsc.VectorSubcoreMesh(core_axis_name="c", subcore_axis_name="s")
    sc = pl.kernel(
        _sc_kernel,
        out_type=[
            jax.ShapeDtypeStruct((_NW, 16), jnp.float32),
            jax.ShapeDtypeStruct((_NW, 16), jnp.float32),
        ],
        mesh=mesh,
        scratch_types=[
            pltpu.VMEM((_CHR, 4), jnp.float32),
            pltpu.VMEM((_CHR, 4), jnp.float32),
            pltpu.VMEM((_CHR,), jnp.int32),
            pltpu.VMEM((16,), jnp.float32),
        ],
        compiler_params=pltpu.CompilerParams(
            use_tc_tiling_on_sc=False, needs_layout_passes=False
        ),
    )
    tot, cnt = sc(out_bbox, bbox_targets, labels)

    out = pl.pallas_call(
        _finish_kernel,
        out_specs=pl.BlockSpec(memory_space=pltpu.SMEM),
        out_shape=jax.ShapeDtypeStruct((1,), jnp.float32),
    )(tot, cnt)
    return out[0]


# speedup vs baseline: 1.2077x; 1.2077x over previous
import functools

import jax
import jax.numpy as jnp
from jax import lax
from jax.experimental import pallas as pl
from jax.experimental.pallas import tpu as pltpu
from jax.experimental.pallas import tpu_sc as plsc

_N = 2097152
_E = _N * 4
_NW = 32
_RPW = _N // _NW
_EPW = _E // _NW
_CHR = 2048
_CHE = _CHR * 4
_CHUNKS = _RPW // _CHR


def _sc_kernel(o_hbm, t_hbm, l_hbm, tot_hbm, cnt_hbm,
               o_buf, t_buf, l_buf, stage):
    wid = lax.axis_index("s") * 2 + lax.axis_index("c")
    row0 = wid * _RPW
    elem0 = wid * _EPW

    lane = lax.iota(jnp.int32, 16)

    def chunk_body(c, carry):
        acc, cnt = carry
        r0 = row0 + c * _CHR
        pltpu.sync_copy(o_hbm.at[pl.ds(r0 * 4, _CHE)], o_buf)
        pltpu.sync_copy(t_hbm.at[pl.ds(r0 * 4, _CHE)], t_buf)
        pltpu.sync_copy(l_hbm.at[pl.ds(r0, _CHR)], l_buf)

        def group_body(g, carry2):
            acc2, cnt2 = carry2
            gr = g * 16
            l16 = l_buf[pl.ds(gr, 16)]
            msk = jnp.where(l16 == 1, 1.0, 0.0)
            eidx = lax.shift_left(gr + lane, 2)
            hsum = jnp.zeros((16,), jnp.float32)
            for col in range(4):
                o_v = plsc.load_gather(o_buf, [eidx + col])
                t_v = plsc.load_gather(t_buf, [eidx + col])
                d = o_v - t_v
                a = jnp.abs(d)
                q = jnp.minimum(a, 1.0)
                hsum = hsum + q * (a - 0.5 * q)
            acc2 = acc2 + msk * hsum
            cnt2 = cnt2 + msk
            return acc2, cnt2

        return lax.fori_loop(0, _CHR // 16, group_body, (acc, cnt))

    zeros = jnp.zeros((16,), jnp.float32)
    acc, cnt = lax.fori_loop(0, _CHUNKS, chunk_body, (zeros, zeros))

    stage[...] = acc
    pltpu.sync_copy(stage, tot_hbm.at[wid])
    stage[...] = cnt
    pltpu.sync_copy(stage, cnt_hbm.at[wid])


def _finish_kernel(tot_ref, cnt_ref, out_ref):
    total = jnp.sum(tot_ref[...])
    npos = jnp.sum(cnt_ref[...])
    out_ref[0] = total / jnp.maximum(npos * 4.0, 1.0)


@jax.jit
def kernel(out_bbox, labels, bbox_targets):
    mesh = plsc.VectorSubcoreMesh(core_axis_name="c", subcore_axis_name="s")
    sc = pl.kernel(
        _sc_kernel,
        out_type=[
            jax.ShapeDtypeStruct((_NW, 16), jnp.float32),
            jax.ShapeDtypeStruct((_NW, 16), jnp.float32),
        ],
        mesh=mesh,
        scratch_types=[
            pltpu.VMEM((_CHE,), jnp.float32),
            pltpu.VMEM((_CHE,), jnp.float32),
            pltpu.VMEM((_CHR,), jnp.int32),
            pltpu.VMEM((16,), jnp.float32),
        ],
        compiler_params=pltpu.CompilerParams(
            use_tc_tiling_on_sc=False, needs_layout_passes=False
        ),
    )
    tot, cnt = sc(out_bbox.reshape(-1), bbox_targets.reshape(-1), labels)

    out = pl.pallas_call(
        _finish_kernel,
        out_specs=pl.BlockSpec(memory_space=pltpu.SMEM),
        out_shape=jax.ShapeDtypeStruct((1,), jnp.float32),
    )(tot, cnt)
    return out[0]


# speedup vs baseline: 47.9742x; 39.7245x over previous
import functools

import jax
import jax.numpy as jnp
from jax import lax
from jax.experimental import pallas as pl
from jax.experimental.pallas import tpu as pltpu
from jax.experimental.pallas import tpu_sc as plsc

_N = 2097152
_E = _N * 4
_NW = 32
_RPW = _N // _NW
_CHR = 2048
_CHE = _CHR * 4
_CHUNKS = _RPW // _CHR
_GROUPS = _CHR // 16


def _sc_kernel(o_hbm, t_hbm, l_hbm, tot_hbm, cnt_hbm,
               o_buf, t_buf, l_buf, stage):
    wid = lax.axis_index("s") * 2 + lax.axis_index("c")
    row0 = wid * _RPW

    def chunk_body(c, carry):
        acc, cnt = carry
        r0 = row0 + c * _CHR
        pltpu.sync_copy(o_hbm.at[pl.ds(r0 * 4, _CHE)], o_buf)
        pltpu.sync_copy(t_hbm.at[pl.ds(r0 * 4, _CHE)], t_buf)
        pltpu.sync_copy(l_hbm.at[pl.ds(r0, _CHR)], l_buf)

        def group_body(g, carry2):
            acc2, cnt2 = carry2
            l16 = l_buf[pl.ds(g * 16, 16)]
            msk = jnp.where(l16 == 1, 1.0, 0.0)
            off = (g >> 3) * 512 + (g & 7) * 16
            hsum = jnp.zeros((16,), jnp.float32)
            for col in range(4):
                o_v = o_buf[pl.ds(off + col * 128, 16)]
                t_v = t_buf[pl.ds(off + col * 128, 16)]
                d = o_v - t_v
                a = jnp.abs(d)
                q = jnp.minimum(a, 1.0)
                hsum = hsum + q * (a - 0.5 * q)
            acc2 = acc2 + msk * hsum
            cnt2 = cnt2 + msk
            return acc2, cnt2

        return lax.fori_loop(0, _GROUPS, group_body, (acc, cnt))

    zeros = jnp.zeros((16,), jnp.float32)
    acc, cnt = lax.fori_loop(0, _CHUNKS, chunk_body, (zeros, zeros))

    stage[...] = acc
    pltpu.sync_copy(stage, tot_hbm.at[wid])
    stage[...] = cnt
    pltpu.sync_copy(stage, cnt_hbm.at[wid])


def _finish_kernel(tot_ref, cnt_ref, out_ref):
    total = jnp.sum(tot_ref[...])
    npos = jnp.sum(cnt_ref[...])
    out_ref[0] = total / jnp.maximum(npos * 4.0, 1.0)


@jax.jit
def kernel(out_bbox, labels, bbox_targets):
    o_flat = out_bbox.reshape(_N // 128, 128, 4).transpose(0, 2, 1).reshape(_E)
    t_flat = (
        bbox_targets.reshape(_N // 128, 128, 4).transpose(0, 2, 1).reshape(_E)
    )

    mesh = plsc.VectorSubcoreMesh(core_axis_name="c", subcore_axis_name="s")
    sc = pl.kernel(
        _sc_kernel,
        out_type=[
            jax.ShapeDtypeStruct((_NW, 16), jnp.float32),
            jax.ShapeDtypeStruct((_NW, 16), jnp.float32),
        ],
        mesh=mesh,
        scratch_types=[
            pltpu.VMEM((_CHE,), jnp.float32),
            pltpu.VMEM((_CHE,), jnp.float32),
            pltpu.VMEM((_CHR,), jnp.int32),
            pltpu.VMEM((16,), jnp.float32),
        ],
        compiler_params=pltpu.CompilerParams(
            use_tc_tiling_on_sc=False, needs_layout_passes=False
        ),
    )
    tot, cnt = sc(o_flat, t_flat, labels)

    out = pl.pallas_call(
        _finish_kernel,
        out_specs=pl.BlockSpec(memory_space=pltpu.SMEM),
        out_shape=jax.ShapeDtypeStruct((1,), jnp.float32),
    )(tot, cnt)
    return out[0]


# speedup vs baseline: 111.2381x; 2.3187x over previous
import functools

import jax
import jax.numpy as jnp
from jax import lax
from jax.experimental import pallas as pl
from jax.experimental.pallas import tpu as pltpu
from jax.experimental.pallas import tpu_sc as plsc

_N = 2097152
_E = _N * 4
_NW = 32
_RPW = _N // _NW
_CHR = 4096
_CHE = _CHR * 4
_CHUNKS = _RPW // _CHR
_GROUPS = _CHR // 16


def _sc_kernel(o_hbm, t_hbm, l_hbm, tot_hbm, cnt_hbm,
               o_bufs, t_bufs, l_bufs, sems, stage):
    wid = lax.axis_index("s") * 2 + lax.axis_index("c")
    row0 = wid * _RPW

    def start(c, b):
        r0 = row0 + c * _CHR
        pltpu.async_copy(o_hbm.at[pl.ds(r0 * 4, _CHE)], o_bufs[b], sems[b])
        pltpu.async_copy(t_hbm.at[pl.ds(r0 * 4, _CHE)], t_bufs[b], sems[b])
        pltpu.async_copy(l_hbm.at[pl.ds(r0, _CHR)], l_bufs[b], sems[b])

    def wait(c, b):
        r0 = row0 + c * _CHR
        pltpu.make_async_copy(
            o_hbm.at[pl.ds(r0 * 4, _CHE)], o_bufs[b], sems[b]
        ).wait()
        pltpu.make_async_copy(
            t_hbm.at[pl.ds(r0 * 4, _CHE)], t_bufs[b], sems[b]
        ).wait()
        pltpu.make_async_copy(
            l_hbm.at[pl.ds(r0, _CHR)], l_bufs[b], sems[b]
        ).wait()

    def compute(b, acc, cnt):
        o_buf, t_buf, l_buf = o_bufs[b], t_bufs[b], l_bufs[b]

        def group_body(g, carry2):
            acc2, cnt2 = carry2
            l16 = l_buf[pl.ds(g * 16, 16)]
            msk = jnp.where(l16 == 1, 1.0, 0.0)
            off = (g >> 3) * 512 + (g & 7) * 16
            hsum = jnp.zeros((16,), jnp.float32)
            for col in range(4):
                o_v = o_buf[pl.ds(off + col * 128, 16)]
                t_v = t_buf[pl.ds(off + col * 128, 16)]
                d = o_v - t_v
                a = jnp.abs(d)
                q = jnp.minimum(a, 1.0)
                hsum = hsum + q * (a - 0.5 * q)
            acc2 = acc2 + msk * hsum
            cnt2 = cnt2 + msk
            return acc2, cnt2

        return lax.fori_loop(0, _GROUPS, group_body, (acc, cnt))

    start(0, 0)
    zeros = jnp.zeros((16,), jnp.float32)

    def pair_body(i, carry):
        acc, cnt = carry
        c0 = i * 2
        start(c0 + 1, 1)
        wait(c0, 0)
        acc, cnt = compute(0, acc, cnt)

        @pl.when(c0 + 2 < _CHUNKS)
        def _():
            start(c0 + 2, 0)

        wait(c0 + 1, 1)
        return compute(1, acc, cnt)

    acc, cnt = lax.fori_loop(0, _CHUNKS // 2, pair_body, (zeros, zeros))

    stage[...] = acc
    pltpu.sync_copy(stage, tot_hbm.at[wid])
    stage[...] = cnt
    pltpu.sync_copy(stage, cnt_hbm.at[wid])


def _finish_kernel(tot_ref, cnt_ref, out_ref):
    total = jnp.sum(tot_ref[...])
    npos = jnp.sum(cnt_ref[...])
    out_ref[0] = total / jnp.maximum(npos * 4.0, 1.0)


@jax.jit
def kernel(out_bbox, labels, bbox_targets):
    o_flat = out_bbox.reshape(_N // 128, 128, 4).transpose(0, 2, 1).reshape(_E)
    t_flat = (
        bbox_targets.reshape(_N // 128, 128, 4).transpose(0, 2, 1).reshape(_E)
    )

    mesh = plsc.VectorSubcoreMesh(core_axis_name="c", subcore_axis_name="s")
    sc = pl.kernel(
        _sc_kernel,
        out_type=[
            jax.ShapeDtypeStruct((_NW, 16), jnp.float32),
            jax.ShapeDtypeStruct((_NW, 16), jnp.float32),
        ],
        mesh=mesh,
        scratch_types=[
            [pltpu.VMEM((_CHE,), jnp.float32)] * 2,
            [pltpu.VMEM((_CHE,), jnp.float32)] * 2,
            [pltpu.VMEM((_CHR,), jnp.int32)] * 2,
            [pltpu.SemaphoreType.DMA] * 2,
            pltpu.VMEM((16,), jnp.float32),
        ],
        compiler_params=pltpu.CompilerParams(
            use_tc_tiling_on_sc=False, needs_layout_passes=False
        ),
    )
    tot, cnt = sc(o_flat, t_flat, labels)

    out = pl.pallas_call(
        _finish_kernel,
        out_specs=pl.BlockSpec(memory_space=pltpu.SMEM),
        out_shape=jax.ShapeDtypeStruct((1,), jnp.float32),
    )(tot, cnt)
    return out[0]
